# baseline (speedup 1.0000x reference)
import functools

import jax
import jax.numpy as jnp
from jax import lax
from jax.experimental import pallas as pl
from jax.experimental.pallas import tpu as pltpu
from jax.experimental.pallas import tpu_sc as plsc

N_USERS = 1000000
N_ITEMS = 1000000
EMBED_DIM = 32
BATCH = 16384

NC = 2
NS = 16
L = 16
NW = NC * NS
B_PER_W = BATCH // NW
IDX_CHUNK = 128
N_CHUNKS = B_PER_W // IDX_CHUNK
G_PER_CHUNK = IDX_CHUNK // L
ROWS_PER_SUPER = 128 // EMBED_DIM


def _mf_kernel(user_idx_hbm, item_idx_hbm, user_table, item_table,
               user_bias, item_bias, gb_hbm, out_hbm,
               idx_u, idx_i, sup_u, sup_i, rows_u, rows_i,
               bias_u, bias_i, gb_v, out_v, sem, sem_b):
    wid = lax.axis_index("s") * NC + lax.axis_index("c")
    base_blk = wid * N_CHUNKS

    pltpu.sync_copy(user_idx_hbm.at[pl.ds(base_blk, N_CHUNKS)], idx_u)
    pltpu.sync_copy(item_idx_hbm.at[pl.ds(base_blk, N_CHUNKS)], idx_i)
    pltpu.sync_copy(gb_hbm, gb_v)

    for j in range(N_CHUNKS):
        for g in range(G_PER_CHUNK):
            s = pl.ds(g * L, L)
            sup_u[j, s] = idx_u[j, s] >> 2
            sup_i[j, s] = idx_i[j, s] >> 2

    bias_copies = []
    for j in range(N_CHUNKS):
        s = pl.ds(j * IDX_CHUNK, IDX_CHUNK)
        bias_copies.append(pltpu.async_copy(
            user_bias.at[idx_u.at[j]], bias_u.at[s], sem_b))
        bias_copies.append(pltpu.async_copy(
            item_bias.at[idx_i.at[j]], bias_i.at[s], sem_b))
    for c in bias_copies:
        c.wait()

    gb = gb_v[...]
    lane = lax.iota(jnp.int32, L)

    for j in range(N_CHUNKS):
        cu = pltpu.async_copy(user_table.at[sup_u.at[j]], rows_u, sem)
        ci = pltpu.async_copy(item_table.at[sup_i.at[j]], rows_i, sem)
        cu.wait()
        ci.wait()

        def body(g, _, j=j):
            row_idx = lane + g * L
            s = pl.ds(j * IDX_CHUNK + g * L, L)
            cb_u = (idx_u[j, pl.ds(g * L, L)] & 3) << 5
            cb_i = (idx_i[j, pl.ds(g * L, L)] & 3) << 5
            acc = None
            for d in range(EMBED_DIM):
                u = plsc.load_gather(rows_u, [row_idx, cb_u + d])
                it = plsc.load_gather(rows_i, [row_idx, cb_i + d])
                acc = u * it if acc is None else acc + u * it
            p = acc + bias_u[s] + bias_i[s] + gb
            out_v[s] = 1.0 / (1.0 + jnp.exp(-p))
            return _

        lax.fori_loop(0, G_PER_CHUNK, body, None)

    pltpu.sync_copy(out_v, out_hbm.at[pl.ds(wid * B_PER_W, B_PER_W)])


def kernel(user_idx, item_idx, user_table, item_table, user_bias_table,
           item_bias_table, global_bias):
    mesh = plsc.VectorSubcoreMesh(core_axis_name="c", subcore_axis_name="s")
    run = pl.kernel(
        _mf_kernel,
        mesh=mesh,
        compiler_params=pltpu.CompilerParams(needs_layout_passes=False),
        out_type=jax.ShapeDtypeStruct((BATCH,), jnp.float32),
        scratch_types=[
            pltpu.VMEM((N_CHUNKS, IDX_CHUNK), jnp.int32),
            pltpu.VMEM((N_CHUNKS, IDX_CHUNK), jnp.int32),
            pltpu.VMEM((N_CHUNKS, IDX_CHUNK), jnp.int32),
            pltpu.VMEM((N_CHUNKS, IDX_CHUNK), jnp.int32),
            pltpu.VMEM((IDX_CHUNK, 128), jnp.float32),
            pltpu.VMEM((IDX_CHUNK, 128), jnp.float32),
            pltpu.VMEM((B_PER_W,), jnp.float32),
            pltpu.VMEM((B_PER_W,), jnp.float32),
            pltpu.VMEM((L,), jnp.float32),
            pltpu.VMEM((B_PER_W,), jnp.float32),
            pltpu.SemaphoreType.DMA,
            pltpu.SemaphoreType.DMA,
        ],
    )
    uidx = user_idx.astype(jnp.int32).reshape(BATCH // IDX_CHUNK, IDX_CHUNK)
    iidx = item_idx.astype(jnp.int32).reshape(BATCH // IDX_CHUNK, IDX_CHUNK)
    ut = user_table.reshape(N_USERS // ROWS_PER_SUPER, 128)
    it = item_table.reshape(N_ITEMS // ROWS_PER_SUPER, 128)
    gb16 = jnp.broadcast_to(global_bias.astype(jnp.float32), (L,))
    return run(uidx, iidx, ut, it,
               user_bias_table.reshape(N_USERS),
               item_bias_table.reshape(N_ITEMS),
               gb16)
